# baseline (speedup 1.0000x reference)
import jax
import jax.numpy as jnp
from jax.experimental import pallas as pl

DIM = 1024
NUM_EXPERTS = 8
HIDDEN = DIM * 2
TOKENS = 4096
BT = 256


def _moe_block(x_ref, gate_w_ref, gate_b_ref, w1_ref, b1_ref, w2_ref, b2_ref,
               out_ref):
    e = pl.program_id(1)
    x = x_ref[...]

    logits = jax.lax.dot_general(
        x, gate_w_ref[...], (((1,), (0,)), ((), ())),
        precision=jax.lax.Precision.DEFAULT,
        preferred_element_type=jnp.float32) + gate_b_ref[...]
    i1 = jnp.argmax(logits, axis=-1, keepdims=True)
    m1 = jnp.max(logits, axis=-1, keepdims=True)
    eidx = jax.lax.broadcasted_iota(jnp.int32, logits.shape, 1)
    masked = jnp.where(eidx == i1, -jnp.inf, logits)
    i2 = jnp.argmax(masked, axis=-1, keepdims=True)
    m2 = jnp.max(masked, axis=-1, keepdims=True)
    p1 = 1.0 / (1.0 + jnp.exp(m2 - m1))
    p2 = 1.0 - p1
    w_e = jnp.where(i1 == e, p1, 0.0) + jnp.where(i2 == e, p2, 0.0)

    h = jax.lax.dot_general(
        x, w1_ref[0], (((1,), (0,)), ((), ())),
        precision=jax.lax.Precision.HIGHEST,
        preferred_element_type=jnp.float32) + b1_ref[0]
    h = 0.5 * h * (1.0 + jax.lax.erf(h * 0.7071067811865476))
    y = jax.lax.dot_general(
        h, w2_ref[0], (((1,), (0,)), ((), ())),
        precision=jax.lax.Precision.HIGHEST,
        preferred_element_type=jnp.float32) + b2_ref[0]
    contrib = y * w_e

    @pl.when(e == 0)
    def _init():
        out_ref[...] = contrib

    @pl.when(e != 0)
    def _acc():
        out_ref[...] += contrib


@jax.jit
def _moe(x, gate_w, gate_b, w1, b1, w2, b2):
    nt = TOKENS // BT
    grid = (nt, NUM_EXPERTS)
    return pl.pallas_call(
        _moe_block,
        grid=grid,
        in_specs=[
            pl.BlockSpec((BT, DIM), lambda i, e: (i, 0)),
            pl.BlockSpec((DIM, NUM_EXPERTS), lambda i, e: (0, 0)),
            pl.BlockSpec((1, NUM_EXPERTS), lambda i, e: (0, 0)),
            pl.BlockSpec((1, DIM, HIDDEN), lambda i, e: (e, 0, 0)),
            pl.BlockSpec((1, 1, HIDDEN), lambda i, e: (e, 0, 0)),
            pl.BlockSpec((1, HIDDEN, DIM), lambda i, e: (e, 0, 0)),
            pl.BlockSpec((1, 1, DIM), lambda i, e: (e, 0, 0)),
        ],
        out_specs=pl.BlockSpec((BT, DIM), lambda i, e: (i, 0)),
        out_shape=jax.ShapeDtypeStruct((TOKENS, DIM), jnp.float32),
    )(x, gate_w, gate_b.reshape(1, NUM_EXPERTS), w1,
      b1.reshape(NUM_EXPERTS, 1, HIDDEN), w2,
      b2.reshape(NUM_EXPERTS, 1, DIM))


def kernel(x, gate_w, gate_b, w1, b1, w2, b2):
    return _moe(x, gate_w, gate_b, w1, b1, w2, b2)


# speedup vs baseline: 2.8253x; 2.8253x over previous
import jax
import jax.numpy as jnp
from jax.experimental import pallas as pl

DIM = 1024
NUM_EXPERTS = 8
HIDDEN = DIM * 2
TOKENS = 4096
BT = 256


def _moe_block(x_ref, gate_w_ref, gate_b_ref, w1_ref, b1_ref, w2_ref, b2_ref,
               out_ref):
    e = pl.program_id(1)
    x = x_ref[...]

    logits = jax.lax.dot_general(
        x, gate_w_ref[...], (((1,), (0,)), ((), ())),
        precision=jax.lax.Precision.DEFAULT,
        preferred_element_type=jnp.float32) + gate_b_ref[...]
    i1 = jnp.argmax(logits, axis=-1, keepdims=True)
    m1 = jnp.max(logits, axis=-1, keepdims=True)
    eidx = jax.lax.broadcasted_iota(jnp.int32, logits.shape, 1)
    masked = jnp.where(eidx == i1, -jnp.inf, logits)
    i2 = jnp.argmax(masked, axis=-1, keepdims=True)
    m2 = jnp.max(masked, axis=-1, keepdims=True)
    p1 = 1.0 / (1.0 + jnp.exp(m2 - m1))
    p2 = 1.0 - p1
    w_e = jnp.where(i1 == e, p1, 0.0) + jnp.where(i2 == e, p2, 0.0)

    h = jax.lax.dot_general(
        x.astype(jnp.bfloat16), w1_ref[0].astype(jnp.bfloat16),
        (((1,), (0,)), ((), ())),
        preferred_element_type=jnp.float32) + b1_ref[0]
    h = 0.5 * h * (1.0 + jax.lax.erf(h * 0.7071067811865476))
    y = jax.lax.dot_general(
        h.astype(jnp.bfloat16), w2_ref[0].astype(jnp.bfloat16),
        (((1,), (0,)), ((), ())),
        preferred_element_type=jnp.float32) + b2_ref[0]
    contrib = y * w_e

    @pl.when(e == 0)
    def _init():
        out_ref[...] = contrib

    @pl.when(e != 0)
    def _acc():
        out_ref[...] += contrib


@jax.jit
def _moe(x, gate_w, gate_b, w1, b1, w2, b2):
    nt = TOKENS // BT
    grid = (nt, NUM_EXPERTS)
    return pl.pallas_call(
        _moe_block,
        grid=grid,
        in_specs=[
            pl.BlockSpec((BT, DIM), lambda i, e: (i, 0)),
            pl.BlockSpec((DIM, NUM_EXPERTS), lambda i, e: (0, 0)),
            pl.BlockSpec((1, NUM_EXPERTS), lambda i, e: (0, 0)),
            pl.BlockSpec((1, DIM, HIDDEN), lambda i, e: (e, 0, 0)),
            pl.BlockSpec((1, 1, HIDDEN), lambda i, e: (e, 0, 0)),
            pl.BlockSpec((1, HIDDEN, DIM), lambda i, e: (e, 0, 0)),
            pl.BlockSpec((1, 1, DIM), lambda i, e: (e, 0, 0)),
        ],
        out_specs=pl.BlockSpec((BT, DIM), lambda i, e: (i, 0)),
        out_shape=jax.ShapeDtypeStruct((TOKENS, DIM), jnp.float32),
    )(x, gate_w, gate_b.reshape(1, NUM_EXPERTS), w1,
      b1.reshape(NUM_EXPERTS, 1, HIDDEN), w2,
      b2.reshape(NUM_EXPERTS, 1, DIM))


def kernel(x, gate_w, gate_b, w1, b1, w2, b2):
    return _moe(x, gate_w, gate_b, w1, b1, w2, b2)


# speedup vs baseline: 3.2422x; 1.1475x over previous
import functools
import jax
import jax.numpy as jnp
from jax import lax
from jax.experimental import pallas as pl
from jax.experimental.pallas import tpu as pltpu
from jax.experimental.pallas import tpu_sc as plsc

DIM = 1024
NUM_EXPERTS = 8
HIDDEN = DIM * 2
TOKENS = 4096
SLOTS = 2 * TOKENS

BT = 256
NB = SLOTS // BT + NUM_EXPERTS
NPAD = NB * BT

NWORKERS = 32
GCHUNK = 64


def _gate_body(x_ref, gw_ref, gb_ref, e1_ref, e2_ref, p1_ref, p2_ref):
    logits = lax.dot_general(
        x_ref[...], gw_ref[...], (((1,), (0,)), ((), ())),
        precision=lax.Precision.DEFAULT,
        preferred_element_type=jnp.float32) + gb_ref[...]
    i1 = jnp.argmax(logits, axis=-1, keepdims=True)
    m1 = jnp.max(logits, axis=-1, keepdims=True)
    eidx = lax.broadcasted_iota(jnp.int32, logits.shape, 1)
    masked = jnp.where(eidx == i1, -jnp.inf, logits)
    i2 = jnp.argmax(masked, axis=-1, keepdims=True)
    m2 = jnp.max(masked, axis=-1, keepdims=True)
    p1 = 1.0 / (1.0 + jnp.exp(m2 - m1))
    e1_ref[...] = i1.astype(jnp.int32)
    e2_ref[...] = i2.astype(jnp.int32)
    p1_ref[...] = p1
    p2_ref[...] = 1.0 - p1


def _gate(x, gate_w, gate_b):
    bt = 1024
    return pl.pallas_call(
        _gate_body,
        grid=(TOKENS // bt,),
        in_specs=[
            pl.BlockSpec((bt, DIM), lambda i: (i, 0)),
            pl.BlockSpec((DIM, NUM_EXPERTS), lambda i: (0, 0)),
            pl.BlockSpec((1, NUM_EXPERTS), lambda i: (0, 0)),
        ],
        out_specs=[
            pl.BlockSpec((bt, 1), lambda i: (i, 0)),
            pl.BlockSpec((bt, 1), lambda i: (i, 0)),
            pl.BlockSpec((bt, 1), lambda i: (i, 0)),
            pl.BlockSpec((bt, 1), lambda i: (i, 0)),
        ],
        out_shape=[
            jax.ShapeDtypeStruct((TOKENS, 1), jnp.int32),
            jax.ShapeDtypeStruct((TOKENS, 1), jnp.int32),
            jax.ShapeDtypeStruct((TOKENS, 1), jnp.float32),
            jax.ShapeDtypeStruct((TOKENS, 1), jnp.float32),
        ],
    )(x, gate_w, gate_b.reshape(1, NUM_EXPERTS))


@functools.lru_cache(maxsize=None)
def _make_sc_gather(n_rows, dim, dtype):
    rows_per_w = n_rows // NWORKERS
    n_chunks = rows_per_w // GCHUNK
    mesh = plsc.VectorSubcoreMesh(core_axis_name="c", subcore_axis_name="s")

    @functools.partial(
        pl.kernel, mesh=mesh,
        out_type=jax.ShapeDtypeStruct((n_rows, dim), dtype),
        scratch_types=[
            pltpu.VMEM((GCHUNK,), jnp.int32),
            pltpu.VMEM((GCHUNK, dim), dtype),
            pltpu.SemaphoreType.DMA,
        ],
    )
    def gather_k(table_hbm, idx_hbm, out_hbm, idx_v, rows_v, sem):
        wid = lax.axis_index("s") * 2 + lax.axis_index("c")
        base = wid * rows_per_w
        for c in range(n_chunks):
            lo = base + c * GCHUNK
            pltpu.sync_copy(idx_hbm.at[pl.ds(lo, GCHUNK)], idx_v)
            pltpu.async_copy(table_hbm.at[idx_v], rows_v, sem).wait()
            pltpu.sync_copy(rows_v, out_hbm.at[pl.ds(lo, GCHUNK)])

    return gather_k


def _sc_gather_xg(table, idx):
    return _make_sc_gather(NPAD, DIM, jnp.float32)(table, idx)


def _sc_gather_y(table, idx):
    return _make_sc_gather(SLOTS, DIM, jnp.float32)(table, idx)


def _ffn_body(be_ref, xg_ref, w1_ref, b1_ref, w2_ref, b2_ref, wp_ref, y_ref):
    del be_ref
    xb = xg_ref[...].astype(jnp.bfloat16)
    h = lax.dot_general(
        xb, w1_ref[0], (((1,), (0,)), ((), ())),
        preferred_element_type=jnp.float32) + b1_ref[0]
    h = 0.5 * h * (1.0 + lax.erf(h * 0.7071067811865476))
    y = lax.dot_general(
        h.astype(jnp.bfloat16), w2_ref[0], (((1,), (0,)), ((), ())),
        preferred_element_type=jnp.float32) + b2_ref[0]
    y_ref[...] = y * wp_ref[0]


def _ffn(block_expert, xg, w1, b1, w2, b2, w_pad):
    grid_spec = pltpu.PrefetchScalarGridSpec(
        num_scalar_prefetch=1,
        grid=(NB,),
        in_specs=[
            pl.BlockSpec((BT, DIM), lambda b, be: (b, 0)),
            pl.BlockSpec((1, DIM, HIDDEN), lambda b, be: (be[b], 0, 0)),
            pl.BlockSpec((1, 1, HIDDEN), lambda b, be: (be[b], 0, 0)),
            pl.BlockSpec((1, HIDDEN, DIM), lambda b, be: (be[b], 0, 0)),
            pl.BlockSpec((1, 1, DIM), lambda b, be: (be[b], 0, 0)),
            pl.BlockSpec((1, BT, 1), lambda b, be: (b, 0, 0)),
        ],
        out_specs=pl.BlockSpec((BT, DIM), lambda b, be: (b, 0)),
    )
    return pl.pallas_call(
        _ffn_body,
        grid_spec=grid_spec,
        out_shape=jax.ShapeDtypeStruct((NPAD, DIM), jnp.float32),
    )(block_expert, xg,
      w1.astype(jnp.bfloat16), b1.reshape(NUM_EXPERTS, 1, HIDDEN),
      w2.astype(jnp.bfloat16), b2.reshape(NUM_EXPERTS, 1, DIM),
      w_pad.reshape(NB, BT, 1))


def _add_body(a_ref, b_ref, o_ref):
    o_ref[...] = a_ref[...] + b_ref[...]


def _combine_add(y01):
    bt = 512
    return pl.pallas_call(
        _add_body,
        grid=(TOKENS // bt,),
        in_specs=[
            pl.BlockSpec((bt, DIM), lambda i: (i, 0)),
            pl.BlockSpec((bt, DIM), lambda i: (i + TOKENS // bt, 0)),
        ],
        out_specs=pl.BlockSpec((bt, DIM), lambda i: (i, 0)),
        out_shape=jax.ShapeDtypeStruct((TOKENS, DIM), jnp.float32),
    )(y01, y01)


@jax.jit
def _moe(x, gate_w, gate_b, w1, b1, w2, b2):
    e1, e2, p1, p2 = _gate(x, gate_w, gate_b)

    e_all = jnp.concatenate([e1[:, 0], e2[:, 0]])
    w_all = jnp.concatenate([p1[:, 0], p2[:, 0]])
    order = jnp.argsort(e_all)
    e_sorted = e_all[order]
    counts = jnp.bincount(e_all, length=NUM_EXPERTS)
    nblk = (counts + BT - 1) // BT
    cumblk = jnp.cumsum(nblk)
    blk_start = jnp.concatenate([jnp.zeros(1, jnp.int32), cumblk[:-1]])
    off = blk_start * BT
    cum_counts = jnp.concatenate([jnp.zeros(1, jnp.int32),
                                  jnp.cumsum(counts)[:-1]])
    q = jnp.arange(SLOTS, dtype=jnp.int32)
    dest = off[e_sorted] + (q - cum_counts[e_sorted])
    tok_pad = jnp.zeros(NPAD, jnp.int32).at[dest].set(order % TOKENS)
    w_pad = jnp.zeros(NPAD, jnp.float32).at[dest].set(w_all[order])
    pos = jnp.zeros(SLOTS, jnp.int32).at[order].set(dest)
    block_expert = jnp.clip(
        jnp.searchsorted(cumblk, jnp.arange(NB, dtype=jnp.int32),
                         side='right').astype(jnp.int32),
        0, NUM_EXPERTS - 1)

    xg = _sc_gather_xg(x, tok_pad)
    y = _ffn(block_expert, xg, w1, b1, w2, b2, w_pad)
    y01 = _sc_gather_y(y, pos)
    return _combine_add(y01)


def kernel(x, gate_w, gate_b, w1, b1, w2, b2):
    return _moe(x, gate_w, gate_b, w1, b1, w2, b2)


# speedup vs baseline: 4.0200x; 1.2399x over previous
import functools
import jax
import jax.numpy as jnp
from jax import lax
from jax.experimental import pallas as pl
from jax.experimental.pallas import tpu as pltpu
from jax.experimental.pallas import tpu_sc as plsc

DIM = 1024
NUM_EXPERTS = 8
HIDDEN = DIM * 2
TOKENS = 4096
SLOTS = 2 * TOKENS

BT = 256
NB = SLOTS // BT + NUM_EXPERTS
NPAD = NB * BT

NWORKERS = 32
GCHUNK = 64


def _gate_body(x_ref, gw_ref, gb_ref, e1_ref, e2_ref, p1_ref, p2_ref):
    logits = lax.dot_general(
        x_ref[...], gw_ref[...], (((1,), (0,)), ((), ())),
        precision=lax.Precision.DEFAULT,
        preferred_element_type=jnp.float32) + gb_ref[...]
    i1 = jnp.argmax(logits, axis=-1, keepdims=True)
    m1 = jnp.max(logits, axis=-1, keepdims=True)
    eidx = lax.broadcasted_iota(jnp.int32, logits.shape, 1)
    masked = jnp.where(eidx == i1, -jnp.inf, logits)
    i2 = jnp.argmax(masked, axis=-1, keepdims=True)
    m2 = jnp.max(masked, axis=-1, keepdims=True)
    p1 = 1.0 / (1.0 + jnp.exp(m2 - m1))
    e1_ref[...] = i1.astype(jnp.int32)
    e2_ref[...] = i2.astype(jnp.int32)
    p1_ref[...] = p1
    p2_ref[...] = 1.0 - p1


def _gate(x, gate_w, gate_b):
    bt = 1024
    return pl.pallas_call(
        _gate_body,
        grid=(TOKENS // bt,),
        in_specs=[
            pl.BlockSpec((bt, DIM), lambda i: (i, 0)),
            pl.BlockSpec((DIM, NUM_EXPERTS), lambda i: (0, 0)),
            pl.BlockSpec((1, NUM_EXPERTS), lambda i: (0, 0)),
        ],
        out_specs=[
            pl.BlockSpec((bt, 1), lambda i: (i, 0)),
            pl.BlockSpec((bt, 1), lambda i: (i, 0)),
            pl.BlockSpec((bt, 1), lambda i: (i, 0)),
            pl.BlockSpec((bt, 1), lambda i: (i, 0)),
        ],
        out_shape=[
            jax.ShapeDtypeStruct((TOKENS, 1), jnp.int32),
            jax.ShapeDtypeStruct((TOKENS, 1), jnp.int32),
            jax.ShapeDtypeStruct((TOKENS, 1), jnp.float32),
            jax.ShapeDtypeStruct((TOKENS, 1), jnp.float32),
        ],
    )(x, gate_w, gate_b.reshape(1, NUM_EXPERTS))


@functools.lru_cache(maxsize=None)
def _make_sc_gather(n_rows, dim, dtype):
    rows_per_w = n_rows // NWORKERS
    n_chunks = rows_per_w // GCHUNK
    mesh = plsc.VectorSubcoreMesh(core_axis_name="c", subcore_axis_name="s")

    @functools.partial(
        pl.kernel, mesh=mesh,
        out_type=jax.ShapeDtypeStruct((n_rows, dim), dtype),
        scratch_types=[
            pltpu.VMEM((GCHUNK,), jnp.int32),
            pltpu.VMEM((GCHUNK, dim), dtype),
            pltpu.SemaphoreType.DMA,
        ],
    )
    def gather_k(table_hbm, idx_hbm, out_hbm, idx_v, rows_v, sem):
        wid = lax.axis_index("s") * 2 + lax.axis_index("c")
        base = wid * rows_per_w
        for c in range(n_chunks):
            lo = base + c * GCHUNK
            pltpu.sync_copy(idx_hbm.at[pl.ds(lo, GCHUNK)], idx_v)
            pltpu.async_copy(table_hbm.at[idx_v], rows_v, sem).wait()
            pltpu.sync_copy(rows_v, out_hbm.at[pl.ds(lo, GCHUNK)])

    return gather_k


def _sc_gather_xg(table, idx):
    return _make_sc_gather(NPAD, DIM, jnp.float32)(table, idx)


def _sc_gather_y(table, idx):
    return _make_sc_gather(SLOTS, DIM, jnp.float32)(table, idx)


def _ffn_body(be_ref, xg_ref, w1_ref, b1_ref, w2_ref, b2_ref, wp_ref, y_ref):
    del be_ref
    xb = xg_ref[...].astype(jnp.bfloat16)
    h = lax.dot_general(
        xb, w1_ref[0], (((1,), (0,)), ((), ())),
        preferred_element_type=jnp.float32) + b1_ref[0]
    h = 0.5 * h * (1.0 + lax.erf(h * 0.7071067811865476))
    y = lax.dot_general(
        h.astype(jnp.bfloat16), w2_ref[0], (((1,), (0,)), ((), ())),
        preferred_element_type=jnp.float32) + b2_ref[0]
    y_ref[...] = y * wp_ref[0]


def _ffn(block_expert, xg, w1, b1, w2, b2, w_pad):
    grid_spec = pltpu.PrefetchScalarGridSpec(
        num_scalar_prefetch=1,
        grid=(NB,),
        in_specs=[
            pl.BlockSpec((BT, DIM), lambda b, be: (b, 0)),
            pl.BlockSpec((1, DIM, HIDDEN), lambda b, be: (be[b], 0, 0)),
            pl.BlockSpec((1, 1, HIDDEN), lambda b, be: (be[b], 0, 0)),
            pl.BlockSpec((1, HIDDEN, DIM), lambda b, be: (be[b], 0, 0)),
            pl.BlockSpec((1, 1, DIM), lambda b, be: (be[b], 0, 0)),
            pl.BlockSpec((1, BT, 1), lambda b, be: (b, 0, 0)),
        ],
        out_specs=pl.BlockSpec((BT, DIM), lambda b, be: (b, 0)),
    )
    return pl.pallas_call(
        _ffn_body,
        grid_spec=grid_spec,
        out_shape=jax.ShapeDtypeStruct((NPAD, DIM), jnp.float32),
    )(block_expert, xg,
      w1.astype(jnp.bfloat16), b1.reshape(NUM_EXPERTS, 1, HIDDEN),
      w2.astype(jnp.bfloat16), b2.reshape(NUM_EXPERTS, 1, DIM),
      w_pad.reshape(NB, BT, 1))


def _add_body(a_ref, b_ref, o_ref):
    o_ref[...] = a_ref[...] + b_ref[...]


def _combine_add(y01):
    bt = 512
    return pl.pallas_call(
        _add_body,
        grid=(TOKENS // bt,),
        in_specs=[
            pl.BlockSpec((bt, DIM), lambda i: (i, 0)),
            pl.BlockSpec((bt, DIM), lambda i: (i + TOKENS // bt, 0)),
        ],
        out_specs=pl.BlockSpec((bt, DIM), lambda i: (i, 0)),
        out_shape=jax.ShapeDtypeStruct((TOKENS, DIM), jnp.float32),
    )(y01, y01)


@jax.jit
def _moe(x, gate_w, gate_b, w1, b1, w2, b2):
    e1, e2, p1, p2 = _gate(x, gate_w, gate_b)

    e_all = jnp.concatenate([e1[:, 0], e2[:, 0]])
    w_all = jnp.concatenate([p1[:, 0], p2[:, 0]])
    order = jnp.argsort(e_all)
    e_sorted = e_all[order]
    counts = jnp.bincount(e_all, length=NUM_EXPERTS)
    nblk = (counts + BT - 1) // BT
    cumblk = jnp.cumsum(nblk)
    blk_start = jnp.concatenate([jnp.zeros(1, jnp.int32), cumblk[:-1]])
    off = blk_start * BT
    cum_counts = jnp.concatenate([jnp.zeros(1, jnp.int32),
                                  jnp.cumsum(counts)[:-1]])
    q = jnp.arange(SLOTS, dtype=jnp.int32)
    dest = off[e_sorted] + (q - cum_counts[e_sorted])
    tok_default = jnp.arange(NPAD, dtype=jnp.int32) % TOKENS
    tok_pad = tok_default.at[dest].set(order % TOKENS)
    w_pad = jnp.zeros(NPAD, jnp.float32).at[dest].set(w_all[order])
    pos = jnp.zeros(SLOTS, jnp.int32).at[order].set(dest)
    block_expert = jnp.clip(
        jnp.searchsorted(cumblk, jnp.arange(NB, dtype=jnp.int32),
                         side='right').astype(jnp.int32),
        0, NUM_EXPERTS - 1)

    xg = _sc_gather_xg(x, tok_pad)
    y = _ffn(block_expert, xg, w1, b1, w2, b2, w_pad)
    y01 = _sc_gather_y(y, pos)
    return _combine_add(y01)


def kernel(x, gate_w, gate_b, w1, b1, w2, b2):
    return _moe(x, gate_w, gate_b, w1, b1, w2, b2)


# speedup vs baseline: 5.7206x; 1.4230x over previous
import functools
import jax
import jax.numpy as jnp
from jax import lax
from jax.experimental import pallas as pl
from jax.experimental.pallas import tpu as pltpu
from jax.experimental.pallas import tpu_sc as plsc

DIM = 1024
NUM_EXPERTS = 8
HIDDEN = DIM * 2
TOKENS = 4096
SLOTS = 2 * TOKENS

BT = 256
NB = SLOTS // BT + NUM_EXPERTS
NPAD = NB * BT
NBPAD = 128

NWORKERS = 32
GCHUNK = 64

GR = 64
GC = 128


def _gate_body(x_ref, gw_ref, gb_ref, e1_ref, e2_ref, p1_ref, p2_ref):
    logits = lax.dot_general(
        gw_ref[...], x_ref[...], (((0,), (1,)), ((), ())),
        precision=lax.Precision.DEFAULT,
        preferred_element_type=jnp.float32) + gb_ref[...]
    i1 = jnp.argmax(logits, axis=0, keepdims=True)
    m1 = jnp.max(logits, axis=0, keepdims=True)
    eidx = lax.broadcasted_iota(jnp.int32, logits.shape, 0)
    masked = jnp.where(eidx == i1, -jnp.inf, logits)
    i2 = jnp.argmax(masked, axis=0, keepdims=True)
    m2 = jnp.max(masked, axis=0, keepdims=True)
    p1 = 1.0 / (1.0 + jnp.exp(m2 - m1))
    e1_ref[0] = i1.astype(jnp.int32)
    e2_ref[0] = i2.astype(jnp.int32)
    p1_ref[0] = p1
    p2_ref[0] = 1.0 - p1


def _gate(x, gate_w, gate_b):
    bt = 1024
    nt = TOKENS // bt
    return pl.pallas_call(
        _gate_body,
        grid=(nt,),
        in_specs=[
            pl.BlockSpec((bt, DIM), lambda i: (i, 0)),
            pl.BlockSpec((DIM, NUM_EXPERTS), lambda i: (0, 0)),
            pl.BlockSpec((NUM_EXPERTS, 1), lambda i: (0, 0)),
        ],
        out_specs=[
            pl.BlockSpec((1, 1, bt), lambda i: (i, 0, 0)),
            pl.BlockSpec((1, 1, bt), lambda i: (i, 0, 0)),
            pl.BlockSpec((1, 1, bt), lambda i: (i, 0, 0)),
            pl.BlockSpec((1, 1, bt), lambda i: (i, 0, 0)),
        ],
        out_shape=[
            jax.ShapeDtypeStruct((nt, 1, bt), jnp.int32),
            jax.ShapeDtypeStruct((nt, 1, bt), jnp.int32),
            jax.ShapeDtypeStruct((nt, 1, bt), jnp.float32),
            jax.ShapeDtypeStruct((nt, 1, bt), jnp.float32),
        ],
    )(x, gate_w, gate_b.reshape(NUM_EXPERTS, 1))


def _routing_body(e1_ref, e2_ref, p1_ref, p2_ref, dest_ref, w_ref, be_ref):
    e1 = jnp.reshape(e1_ref[...], (GR // 2, GC))
    e2 = jnp.reshape(e2_ref[...], (GR // 2, GC))
    ea = jnp.concatenate([e1, e2], axis=0)
    p1 = jnp.reshape(p1_ref[...], (GR // 2, GC))
    p2 = jnp.reshape(p2_ref[...], (GR // 2, GC))
    w_ref[...] = jnp.concatenate([p1, p2], axis=0)

    ci = lax.broadcasted_iota(jnp.int32, (GC, GC), 0)
    cj = lax.broadcasted_iota(jnp.int32, (GC, GC), 1)
    tri_inc = (ci <= cj).astype(jnp.bfloat16)
    ri = lax.broadcasted_iota(jnp.int32, (GR, GR), 0)
    rj = lax.broadcasted_iota(jnp.int32, (GR, GR), 1)
    tri_strict = (ri > rj).astype(jnp.bfloat16)

    dest = jnp.zeros((GR, GC), jnp.float32)
    running = jnp.zeros((), jnp.float32)
    cumblk = []
    for e in range(NUM_EXPERTS):
        m = (ea == e).astype(jnp.float32)
        rowcs = lax.dot_general(m.astype(jnp.bfloat16), tri_inc,
                                (((1,), (0,)), ((), ())),
                                preferred_element_type=jnp.float32)
        rowsum = rowcs[:, GC - 1:GC]
        prevrows = lax.dot_general(tri_strict,
                                   rowsum.astype(jnp.bfloat16),
                                   (((1,), (0,)), ((), ())),
                                   preferred_element_type=jnp.float32)
        rank = rowcs - m + prevrows
        cnt = jnp.sum(m)
        dest = dest + m * (running + rank)
        running = running + jnp.ceil(cnt / BT) * BT
        cumblk.append(running / BT)

    dest_ref[...] = dest.astype(jnp.int32)

    bi = lax.broadcasted_iota(jnp.int32, (1, NBPAD), 1).astype(jnp.float32)
    be = jnp.zeros((1, NBPAD), jnp.int32)
    for e in range(NUM_EXPERTS - 1):
        be = be + (bi >= cumblk[e]).astype(jnp.int32)
    be_ref[...] = be


def _routing(e1, e2, p1, p2):
    nt = TOKENS // 1024
    specs = [pl.BlockSpec((nt, 1, 1024), lambda: (0, 0, 0))] * 4
    return pl.pallas_call(
        _routing_body,
        in_specs=specs,
        out_specs=[
            pl.BlockSpec((GR, GC), lambda: (0, 0)),
            pl.BlockSpec((GR, GC), lambda: (0, 0)),
            pl.BlockSpec((1, NBPAD), lambda: (0, 0)),
        ],
        out_shape=[
            jax.ShapeDtypeStruct((GR, GC), jnp.int32),
            jax.ShapeDtypeStruct((GR, GC), jnp.float32),
            jax.ShapeDtypeStruct((1, NBPAD), jnp.int32),
        ],
    )(e1, e2, p1, p2)


@functools.lru_cache(maxsize=None)
def _make_sc_dispatch():
    slots_per_w = SLOTS // NWORKERS
    n_chunks = slots_per_w // GCHUNK
    mesh = plsc.VectorSubcoreMesh(core_axis_name="c", subcore_axis_name="s")

    @functools.partial(
        pl.kernel, mesh=mesh,
        out_type=[
            jax.ShapeDtypeStruct((NPAD, DIM), jnp.float32),
            jax.ShapeDtypeStruct((NPAD,), jnp.float32),
        ],
        scratch_types=[
            pltpu.VMEM((GCHUNK,), jnp.int32),
            pltpu.VMEM((GCHUNK,), jnp.int32),
            pltpu.VMEM((GCHUNK,), jnp.float32),
            pltpu.VMEM((GCHUNK, DIM), jnp.float32),
            pltpu.SemaphoreType.DMA,
            pltpu.SemaphoreType.DMA,
        ],
    )
    def dispatch_k(x_hbm, dest_hbm, w_hbm, xg_hbm, wpad_hbm,
                   tok_v, dest_v, w_v, rows_v, sem, sem2):
        wid = lax.axis_index("s") * 2 + lax.axis_index("c")
        base = wid * slots_per_w
        for c in range(n_chunks):
            lo = base + c * GCHUNK
            pltpu.sync_copy(dest_hbm.at[pl.ds(lo, GCHUNK)], dest_v)
            pltpu.sync_copy(w_hbm.at[pl.ds(lo, GCHUNK)], w_v)
            for g in range(GCHUNK // 16):
                s16 = lo + g * 16 + lax.broadcasted_iota(jnp.int32, (16,), 0)
                tok_v[pl.ds(g * 16, 16)] = jnp.where(
                    s16 >= TOKENS, s16 - TOKENS, s16)
            pltpu.async_copy(x_hbm.at[tok_v], rows_v, sem).wait()
            pltpu.async_copy(rows_v, xg_hbm.at[dest_v], sem2).wait()
            pltpu.sync_copy(w_v, wpad_hbm.at[dest_v])

    return dispatch_k


def _sc_dispatch(x, dest, w_all):
    return _make_sc_dispatch()(x, dest, w_all)


@functools.lru_cache(maxsize=None)
def _make_sc_gather(n_rows, dim, dtype):
    rows_per_w = n_rows // NWORKERS
    n_chunks = rows_per_w // GCHUNK
    mesh = plsc.VectorSubcoreMesh(core_axis_name="c", subcore_axis_name="s")

    @functools.partial(
        pl.kernel, mesh=mesh,
        out_type=jax.ShapeDtypeStruct((n_rows, dim), dtype),
        scratch_types=[
            pltpu.VMEM((GCHUNK,), jnp.int32),
            pltpu.VMEM((GCHUNK, dim), dtype),
            pltpu.SemaphoreType.DMA,
        ],
    )
    def gather_k(table_hbm, idx_hbm, out_hbm, idx_v, rows_v, sem):
        wid = lax.axis_index("s") * 2 + lax.axis_index("c")
        base = wid * rows_per_w
        for c in range(n_chunks):
            lo = base + c * GCHUNK
            pltpu.sync_copy(idx_hbm.at[pl.ds(lo, GCHUNK)], idx_v)
            pltpu.async_copy(table_hbm.at[idx_v], rows_v, sem).wait()
            pltpu.sync_copy(rows_v, out_hbm.at[pl.ds(lo, GCHUNK)])

    return gather_k


def _sc_gather_y(table, idx):
    return _make_sc_gather(SLOTS, DIM, jnp.float32)(table, idx)


def _ffn_body(be_ref, xg_ref, w1_ref, b1_ref, w2_ref, b2_ref, wp_ref, y_ref):
    del be_ref
    xb = xg_ref[...].astype(jnp.bfloat16)
    h = lax.dot_general(
        xb, w1_ref[0], (((1,), (0,)), ((), ())),
        preferred_element_type=jnp.float32) + b1_ref[0]
    h = 0.5 * h * (1.0 + lax.erf(h * 0.7071067811865476))
    y = lax.dot_general(
        h.astype(jnp.bfloat16), w2_ref[0], (((1,), (0,)), ((), ())),
        preferred_element_type=jnp.float32) + b2_ref[0]
    y_ref[...] = y * wp_ref[0]


def _ffn(block_expert, xg, w1, b1, w2, b2, w_pad):
    grid_spec = pltpu.PrefetchScalarGridSpec(
        num_scalar_prefetch=1,
        grid=(NB,),
        in_specs=[
            pl.BlockSpec((BT, DIM), lambda b, be: (b, 0)),
            pl.BlockSpec((1, DIM, HIDDEN), lambda b, be: (be[b], 0, 0)),
            pl.BlockSpec((1, 1, HIDDEN), lambda b, be: (be[b], 0, 0)),
            pl.BlockSpec((1, HIDDEN, DIM), lambda b, be: (be[b], 0, 0)),
            pl.BlockSpec((1, 1, DIM), lambda b, be: (be[b], 0, 0)),
            pl.BlockSpec((1, BT, 1), lambda b, be: (b, 0, 0)),
        ],
        out_specs=pl.BlockSpec((BT, DIM), lambda b, be: (b, 0)),
    )
    return pl.pallas_call(
        _ffn_body,
        grid_spec=grid_spec,
        out_shape=jax.ShapeDtypeStruct((NPAD, DIM), jnp.float32),
    )(block_expert, xg,
      w1.astype(jnp.bfloat16), b1.reshape(NUM_EXPERTS, 1, HIDDEN),
      w2.astype(jnp.bfloat16), b2.reshape(NUM_EXPERTS, 1, DIM),
      w_pad.reshape(NB, BT, 1))


def _add_body(a_ref, b_ref, o_ref):
    o_ref[...] = a_ref[...] + b_ref[...]


def _combine_add(y01):
    bt = 512
    return pl.pallas_call(
        _add_body,
        grid=(TOKENS // bt,),
        in_specs=[
            pl.BlockSpec((bt, DIM), lambda i: (i, 0)),
            pl.BlockSpec((bt, DIM), lambda i: (i + TOKENS // bt, 0)),
        ],
        out_specs=pl.BlockSpec((bt, DIM), lambda i: (i, 0)),
        out_shape=jax.ShapeDtypeStruct((TOKENS, DIM), jnp.float32),
    )(y01, y01)


@jax.jit
def _moe(x, gate_w, gate_b, w1, b1, w2, b2):
    e1, e2, p1, p2 = _gate(x, gate_w, gate_b)
    dest2d, w2d, bexp = _routing(e1, e2, p1, p2)
    dest = dest2d.reshape(SLOTS)
    w_all = w2d.reshape(SLOTS)
    xg, w_pad = _sc_dispatch(x, dest, w_all)
    y = _ffn(bexp.reshape(NBPAD), xg, w1, b1, w2, b2, w_pad)
    y01 = _sc_gather_y(y, dest)
    return _combine_add(y01)


def kernel(x, gate_w, gate_b, w1, b1, w2, b2):
    return _moe(x, gate_w, gate_b, w1, b1, w2, b2)
